# baseline (speedup 1.0000x reference)
import functools

import jax
import jax.numpy as jnp
from jax import lax
from jax.experimental import pallas as pl
from jax.experimental.pallas import tpu as pltpu
from jax.experimental.pallas import tpu_sc as plsc

N_CORES = 2
N_SUB = 16
NW = N_CORES * N_SUB
CH = 128
PCH = 64


def _sc_mesh():
    return plsc.VectorSubcoreMesh(
        core_axis_name="c", subcore_axis_name="s",
        num_cores=N_CORES, num_subcores=N_SUB)


def _chunks_for(wid, e, chunk=CH):
    n_chunks = e // chunk
    base = n_chunks // NW
    extra = n_chunks % NW
    return jnp.where(wid < extra, base + 1, base)


def _row_share(n):
    rows_sub = ((n + N_SUB - 1) // N_SUB + 7) // 8 * 8
    last = n - rows_sub * (N_SUB - 1)
    assert last > 0 and last % 8 == 0
    return rows_sub, last


@functools.lru_cache(maxsize=None)
def _scatter_add_kernel(n, e, d):
    rows_sub, rows_last = _row_share(n)
    acc_n = rows_sub * N_SUB
    assert e // CH >= 2 * NW

    @functools.partial(
        pl.kernel,
        out_type=jax.ShapeDtypeStruct((N_CORES, n, d), jnp.float32),
        mesh=_sc_mesh(),
        scratch_types=[
            pltpu.VMEM((2, CH), jnp.int32),
            pltpu.VMEM((2, CH), jnp.int32),
            pltpu.VMEM((2, CH, d), jnp.float32),
            pltpu.VMEM_SHARED((acc_n, d), jnp.float32),
            pltpu.SemaphoreType.DMA((2,)),
            pltpu.SemaphoreType.DMA((2,)),
        ],
    )
    def k(src_hbm, dst_hbm, tab_hbm, zeros_hbm, out_hbm,
          idx_s, idx_d, rows_v, acc_sh, sem_i, sem_g):
        c = lax.axis_index("c")
        s = lax.axis_index("s")
        wid = s * N_CORES + c
        r0 = s * rows_sub
        nch = _chunks_for(wid, e)

        def issue_idx(ci, b):
            off = (wid + ci * NW) * CH
            pltpu.async_copy(src_hbm.at[pl.ds(off, CH)], idx_s.at[b], sem_i.at[b])
            pltpu.async_copy(dst_hbm.at[pl.ds(off, CH)], idx_d.at[b], sem_i.at[b])

        def wait_idx(b):
            pltpu.make_async_copy(src_hbm.at[pl.ds(0, CH)], idx_s.at[b],
                                  sem_i.at[b]).wait()
            pltpu.make_async_copy(dst_hbm.at[pl.ds(0, CH)], idx_d.at[b],
                                  sem_i.at[b]).wait()

        def issue_gather(ci, b):
            del ci
            pltpu.async_copy(tab_hbm.at[idx_s.at[b]], rows_v.at[b], sem_g.at[b])

        def wait_gather(b):
            pltpu.make_async_copy(tab_hbm.at[pl.ds(0, CH), :], rows_v.at[b],
                                  sem_g.at[b]).wait()

        issue_idx(0, 0)
        issue_idx(1, 1)
        pltpu.sync_copy(zeros_hbm, acc_sh.at[pl.ds(r0, rows_sub), :])
        plsc.subcore_barrier()
        wait_idx(0)
        issue_gather(0, 0)

        def body(i, _):
            b = lax.rem(i, 2)
            o = 1 - b
            wait_gather(b)

            @pl.when(i + 1 < nch)
            def _():
                wait_idx(o)
                issue_gather(i + 1, o)

            pltpu.sync_copy(rows_v.at[b], acc_sh.at[idx_d.at[b]], add=True)

            @pl.when(i + 2 < nch)
            def _():
                issue_idx(i + 2, b)

            return 0

        lax.fori_loop(0, nch, body, 0)
        plsc.subcore_barrier()

        @pl.when(s < N_SUB - 1)
        def _():
            pltpu.sync_copy(acc_sh.at[pl.ds(r0, rows_sub), :],
                            out_hbm.at[c, pl.ds(r0, rows_sub), :])

        @pl.when(s == N_SUB - 1)
        def _():
            pltpu.sync_copy(acc_sh.at[pl.ds(r0, rows_last), :],
                            out_hbm.at[c, pl.ds(r0, rows_last), :])

    return k


@functools.lru_cache(maxsize=None)
def _deg_kernel(n, e):
    rows_sub, rows_last = _row_share(n)
    acc_n = rows_sub * N_SUB

    @functools.partial(
        pl.kernel,
        out_type=jax.ShapeDtypeStruct((N_CORES, n, 128), jnp.float32),
        mesh=_sc_mesh(),
        scratch_types=[
            pltpu.VMEM((2, CH), jnp.int32),
            pltpu.VMEM((CH, 128), jnp.float32),
            pltpu.VMEM_SHARED((acc_n, 128), jnp.float32),
            pltpu.SemaphoreType.DMA((2,)),
        ],
    )
    def k(dst_hbm, ones_hbm, zeros_hbm, out_hbm, idx_d, ones_v, acc_sh, sem_i):
        c = lax.axis_index("c")
        s = lax.axis_index("s")
        wid = s * N_CORES + c
        r0 = s * rows_sub
        nch = _chunks_for(wid, e)

        def issue_idx(ci, b):
            off = (wid + ci * NW) * CH
            pltpu.async_copy(dst_hbm.at[pl.ds(off, CH)], idx_d.at[b], sem_i.at[b])

        issue_idx(0, 0)
        issue_idx(1, 1)
        pltpu.sync_copy(zeros_hbm, acc_sh.at[pl.ds(r0, rows_sub), :])
        pltpu.sync_copy(ones_hbm, ones_v)
        plsc.subcore_barrier()

        def body(i, _):
            b = lax.rem(i, 2)
            pltpu.make_async_copy(dst_hbm.at[pl.ds(0, CH)], idx_d.at[b],
                                  sem_i.at[b]).wait()
            pltpu.sync_copy(ones_v, acc_sh.at[idx_d.at[b]], add=True)

            @pl.when(i + 2 < nch)
            def _():
                issue_idx(i + 2, b)

            return 0

        lax.fori_loop(0, nch, body, 0)
        plsc.subcore_barrier()

        @pl.when(s < N_SUB - 1)
        def _():
            pltpu.sync_copy(acc_sh.at[pl.ds(r0, rows_sub), :],
                            out_hbm.at[c, pl.ds(r0, rows_sub), :])

        @pl.when(s == N_SUB - 1)
        def _():
            pltpu.sync_copy(acc_sh.at[pl.ds(r0, rows_last), :],
                            out_hbm.at[c, pl.ds(r0, rows_last), :])

    return k


@functools.lru_cache(maxsize=None)
def _pair_sum_kernel(n, e, d):
    rows_sub, rows_last = _row_share(n)
    acc_n = rows_sub * N_SUB
    assert e // PCH >= 2 * NW

    @functools.partial(
        pl.kernel,
        out_type=jax.ShapeDtypeStruct((e, d), jnp.float32),
        mesh=_sc_mesh(),
        scratch_types=[
            pltpu.VMEM((2, PCH), jnp.int32),
            pltpu.VMEM((2, PCH), jnp.int32),
            pltpu.VMEM((2, PCH), jnp.int32),
            pltpu.VMEM((2, PCH, d), jnp.float32),
            pltpu.VMEM((2, PCH, d), jnp.float32),
            pltpu.VMEM_SHARED((N_SUB * 2 * PCH, d), jnp.float32),
            pltpu.VMEM_SHARED((n, d), jnp.float32),
            pltpu.SemaphoreType.DMA((2,)),
            pltpu.SemaphoreType.DMA((2,)),
            pltpu.SemaphoreType.DMA((2,)),
            pltpu.SemaphoreType.DMA((2,)),
        ],
    )
    def k(p0_hbm, p1_hbm, ident_hbm, u_hbm, v_hbm, s_hbm,
          idx0, idx1, ident, gu, gv, s_sh, u_sh, sem_i, sem_gu, sem_gv, sem_st):
        c = lax.axis_index("c")
        s = lax.axis_index("s")
        wid = s * N_CORES + c
        r0 = s * rows_sub
        nch = _chunks_for(wid, e, PCH)

        def region(b):
            return pl.ds((s * 2 + b) * PCH, PCH)

        def issue_idx(ci, b):
            off = (wid + ci * NW) * PCH
            pltpu.async_copy(p0_hbm.at[pl.ds(off, PCH)], idx0.at[b], sem_i.at[b])
            pltpu.async_copy(p1_hbm.at[pl.ds(off, PCH)], idx1.at[b], sem_i.at[b])

        def wait_idx(b):
            pltpu.make_async_copy(p0_hbm.at[pl.ds(0, PCH)], idx0.at[b],
                                  sem_i.at[b]).wait()
            pltpu.make_async_copy(p1_hbm.at[pl.ds(0, PCH)], idx1.at[b],
                                  sem_i.at[b]).wait()

        def issue_gathers(b):
            pltpu.async_copy(u_sh.at[idx0.at[b]], gu.at[b], sem_gu.at[b])
            pltpu.async_copy(v_hbm.at[idx1.at[b]], gv.at[b], sem_gv.at[b])

        def wait_gathers(b):
            pltpu.make_async_copy(u_sh.at[pl.ds(0, PCH), :], gu.at[b],
                                  sem_gu.at[b]).wait()
            pltpu.make_async_copy(v_hbm.at[pl.ds(0, PCH), :], gv.at[b],
                                  sem_gv.at[b]).wait()

        def wait_store(b):
            pltpu.make_async_copy(s_sh.at[region(0), :],
                                  s_hbm.at[pl.ds(0, PCH), :], sem_st.at[b]).wait()

        pltpu.sync_copy(ident_hbm.at[pl.ds(s * 2, 2)], ident)
        issue_idx(0, 0)
        issue_idx(1, 1)

        @pl.when(s < N_SUB - 1)
        def _():
            pltpu.sync_copy(u_hbm.at[pl.ds(r0, rows_sub), :],
                            u_sh.at[pl.ds(r0, rows_sub), :])

        @pl.when(s == N_SUB - 1)
        def _():
            pltpu.sync_copy(u_hbm.at[pl.ds(r0, rows_last), :],
                            u_sh.at[pl.ds(r0, rows_last), :])

        plsc.subcore_barrier()
        wait_idx(0)
        issue_gathers(0)

        def body(i, _):
            b = lax.rem(i, 2)
            o = 1 - b
            off = (wid + i * NW) * PCH
            wait_gathers(b)

            @pl.when(i + 1 < nch)
            def _():
                wait_idx(o)

                @pl.when(i >= 1)
                def _():
                    wait_store(o)

                issue_gathers(o)

            pltpu.sync_copy(gu.at[b], s_sh.at[region(b), :])
            pltpu.sync_copy(gv.at[b], s_sh.at[ident.at[b]], add=True)
            pltpu.async_copy(s_sh.at[region(b), :],
                             s_hbm.at[pl.ds(off, PCH), :], sem_st.at[b])

            @pl.when(i + 2 < nch)
            def _():
                issue_idx(i + 2, b)

            return 0

        lax.fori_loop(0, nch, body, 0)
        b_last = lax.rem(nch - 1, 2)
        wait_store(b_last)
        wait_store(1 - b_last)

    return k


def _dinv_of(deg_ref):
    deg = deg_ref[0, :, 0:1] + deg_ref[1, :, 0:1] + 1.0
    return lax.rsqrt(deg)


def _tc_first(degp, x, w, blk=1000):
    n, dfeat = x.shape
    dout = w.shape[1]

    def body(deg_ref, x_ref, w_ref, y_ref):
        y_ref[...] = jnp.dot(x_ref[...], w_ref[...],
                             preferred_element_type=jnp.float32) * _dinv_of(deg_ref)

    return pl.pallas_call(
        body,
        grid=(n // blk,),
        in_specs=[
            pl.BlockSpec((2, blk, 128), lambda i: (0, i, 0)),
            pl.BlockSpec((blk, dfeat), lambda i: (i, 0)),
            pl.BlockSpec((dfeat, dout), lambda i: (0, 0)),
        ],
        out_specs=pl.BlockSpec((blk, dout), lambda i: (i, 0)),
        out_shape=jax.ShapeDtypeStruct((n, dout), jnp.float32),
    )(degp, x, w)


def _tc_mid(degp, acc, y, b, w, blk=1000):
    n, din = y.shape
    dout = w.shape[1]

    def body(deg_ref, acc_ref, y_ref, b_ref, w_ref, o_ref):
        dinv = _dinv_of(deg_ref)
        h = jnp.maximum(dinv * (acc_ref[0] + acc_ref[1] + y_ref[...]) + b_ref[...], 0.0)
        o_ref[...] = jnp.dot(h, w_ref[...],
                             preferred_element_type=jnp.float32) * dinv

    return pl.pallas_call(
        body,
        grid=(n // blk,),
        in_specs=[
            pl.BlockSpec((2, blk, 128), lambda i: (0, i, 0)),
            pl.BlockSpec((2, blk, din), lambda i: (0, i, 0)),
            pl.BlockSpec((blk, din), lambda i: (i, 0)),
            pl.BlockSpec((1, din), lambda i: (0, 0)),
            pl.BlockSpec((din, dout), lambda i: (0, 0)),
        ],
        out_specs=pl.BlockSpec((blk, dout), lambda i: (i, 0)),
        out_shape=jax.ShapeDtypeStruct((n, dout), jnp.float32),
    )(degp, acc, y, b, w)


def _tc_last_node(degp, acc, y, b, w3a, w3b, blk=1000):
    n, din = y.shape
    dout = w3a.shape[1]

    def body(deg_ref, acc_ref, y_ref, b_ref, wa_ref, wb_ref, u_ref, v_ref):
        dinv = _dinv_of(deg_ref)
        h = jnp.maximum(dinv * (acc_ref[0] + acc_ref[1] + y_ref[...]) + b_ref[...], 0.0)
        u_ref[...] = jnp.dot(h, wa_ref[...], preferred_element_type=jnp.float32)
        v_ref[...] = jnp.dot(h, wb_ref[...], preferred_element_type=jnp.float32)

    return pl.pallas_call(
        body,
        grid=(n // blk,),
        in_specs=[
            pl.BlockSpec((2, blk, 128), lambda i: (0, i, 0)),
            pl.BlockSpec((2, blk, din), lambda i: (0, i, 0)),
            pl.BlockSpec((blk, din), lambda i: (i, 0)),
            pl.BlockSpec((1, din), lambda i: (0, 0)),
            pl.BlockSpec((din, dout), lambda i: (0, 0)),
            pl.BlockSpec((din, dout), lambda i: (0, 0)),
        ],
        out_specs=[
            pl.BlockSpec((blk, dout), lambda i: (i, 0)),
            pl.BlockSpec((blk, dout), lambda i: (i, 0)),
        ],
        out_shape=[jax.ShapeDtypeStruct((n, dout), jnp.float32),
                   jax.ShapeDtypeStruct((n, dout), jnp.float32)],
    )(degp, acc, y, b, w3a, w3b)


def _tc_head(s, b3, w4, b4, blk=2000):
    e, d = s.shape

    def body(s_ref, b3_ref, w4_ref, b4_ref, o_ref):
        hid = jnp.maximum(s_ref[...] + b3_ref[...], 0.0)
        logit = jnp.dot(hid, w4_ref[...], preferred_element_type=jnp.float32)
        o_ref[...] = jax.nn.sigmoid(logit + b4_ref[...])

    return pl.pallas_call(
        body,
        grid=(e // blk,),
        in_specs=[
            pl.BlockSpec((blk, d), lambda i: (i, 0)),
            pl.BlockSpec((1, d), lambda i: (0, 0)),
            pl.BlockSpec((d, 1), lambda i: (0, 0)),
            pl.BlockSpec((1, 1), lambda i: (0, 0)),
        ],
        out_specs=pl.BlockSpec((blk, 1), lambda i: (i, 0)),
        out_shape=jax.ShapeDtypeStruct((e, 1), jnp.float32),
    )(s, b3, w4, b4)


def kernel(x, edge_index, edge_pairs, W1, b1, W2, b2, W3, b3, W4, b4):
    x = x.astype(jnp.float32)
    ei = edge_index.astype(jnp.int32)
    ep = edge_pairs.astype(jnp.int32)
    src, dst = ei[0], ei[1]
    p0, p1 = ep[0], ep[1]
    n = x.shape[0]
    e = src.shape[0]
    rows_sub, _ = _row_share(n)
    d1 = W1.shape[1]
    d2 = W2.shape[1]
    dh = W3.shape[1]

    dp = 128
    W1p = jnp.pad(W1, ((0, 0), (0, dp - d1)))
    b1p = jnp.pad(b1, (0, dp - d1))
    W2p = jnp.pad(W2, ((0, dp - d1), (0, 0)))

    ones16 = jnp.ones((CH, 128), jnp.float32)
    z16 = jnp.zeros((rows_sub, 128), jnp.float32)
    z1 = jnp.zeros((rows_sub, dp), jnp.float32)
    z2 = jnp.zeros((rows_sub, d2), jnp.float32)

    ident = jnp.arange(N_SUB * 2 * PCH, dtype=jnp.int32).reshape(N_SUB * 2, PCH)

    degp = _deg_kernel(n, e)(dst, ones16, z16)
    y1 = _tc_first(degp, x, W1p)
    acc1 = _scatter_add_kernel(n, e, dp)(src, dst, y1, z1)
    y2 = _tc_mid(degp, acc1, y1, b1p.reshape(1, -1), W2p)
    acc2 = _scatter_add_kernel(n, e, d2)(src, dst, y2, z2)
    u, v = _tc_last_node(degp, acc2, y2, b2.reshape(1, -1),
                         W3[:d2], W3[d2:])
    eh = e // 2
    outs = []
    for h in range(2):
        ph0 = lax.dynamic_slice_in_dim(p0, h * eh, eh)
        ph1 = lax.dynamic_slice_in_dim(p1, h * eh, eh)
        s = _pair_sum_kernel(n, eh, dh)(ph0, ph1, ident, u, v)
        outs.append(_tc_head(s, b3.reshape(1, -1), W4, b4.reshape(1, 1)))
    return jnp.concatenate(outs, axis=0)


# speedup vs baseline: 1.0624x; 1.0624x over previous
import functools

import jax
import jax.numpy as jnp
from jax import lax
from jax.experimental import pallas as pl
from jax.experimental.pallas import tpu as pltpu
from jax.experimental.pallas import tpu_sc as plsc

N_CORES = 2
N_SUB = 16
NW = N_CORES * N_SUB
CH = 128


def _sc_mesh():
    return plsc.VectorSubcoreMesh(
        core_axis_name="c", subcore_axis_name="s",
        num_cores=N_CORES, num_subcores=N_SUB)


def _chunks_for(wid, e, chunk=CH):
    n_chunks = e // chunk
    base = n_chunks // NW
    extra = n_chunks % NW
    return jnp.where(wid < extra, base + 1, base)


def _row_share(n):
    rows_sub = ((n + N_SUB - 1) // N_SUB + 7) // 8 * 8
    last = n - rows_sub * (N_SUB - 1)
    assert last > 0 and last % 8 == 0
    return rows_sub, last


@functools.lru_cache(maxsize=None)
def _scatter_add_kernel(n, e, d):
    rows_sub, rows_last = _row_share(n)
    acc_n = rows_sub * N_SUB
    assert e // CH >= 2 * NW

    @functools.partial(
        pl.kernel,
        out_type=jax.ShapeDtypeStruct((N_CORES, n, d), jnp.float32),
        mesh=_sc_mesh(),
        scratch_types=[
            pltpu.VMEM((2, CH), jnp.int32),
            pltpu.VMEM((2, CH), jnp.int32),
            pltpu.VMEM((2, CH, d), jnp.float32),
            pltpu.VMEM_SHARED((acc_n, d), jnp.float32),
            pltpu.SemaphoreType.DMA((2,)),
            pltpu.SemaphoreType.DMA((2,)),
        ],
    )
    def k(src_hbm, dst_hbm, tab_hbm, zeros_hbm, out_hbm,
          idx_s, idx_d, rows_v, acc_sh, sem_i, sem_g):
        c = lax.axis_index("c")
        s = lax.axis_index("s")
        wid = s * N_CORES + c
        r0 = s * rows_sub
        nch = _chunks_for(wid, e)

        def issue_idx(ci, b):
            off = (wid + ci * NW) * CH
            pltpu.async_copy(src_hbm.at[pl.ds(off, CH)], idx_s.at[b], sem_i.at[b])
            pltpu.async_copy(dst_hbm.at[pl.ds(off, CH)], idx_d.at[b], sem_i.at[b])

        def wait_idx(b):
            pltpu.make_async_copy(src_hbm.at[pl.ds(0, CH)], idx_s.at[b],
                                  sem_i.at[b]).wait()
            pltpu.make_async_copy(dst_hbm.at[pl.ds(0, CH)], idx_d.at[b],
                                  sem_i.at[b]).wait()

        def issue_gather(ci, b):
            del ci
            pltpu.async_copy(tab_hbm.at[idx_s.at[b]], rows_v.at[b], sem_g.at[b])

        def wait_gather(b):
            pltpu.make_async_copy(tab_hbm.at[pl.ds(0, CH), :], rows_v.at[b],
                                  sem_g.at[b]).wait()

        issue_idx(0, 0)
        issue_idx(1, 1)
        pltpu.sync_copy(zeros_hbm, acc_sh.at[pl.ds(r0, rows_sub), :])
        plsc.subcore_barrier()
        wait_idx(0)
        issue_gather(0, 0)

        def body(i, _):
            b = lax.rem(i, 2)
            o = 1 - b
            wait_gather(b)

            @pl.when(i + 1 < nch)
            def _():
                wait_idx(o)
                issue_gather(i + 1, o)

            pltpu.sync_copy(rows_v.at[b], acc_sh.at[idx_d.at[b]], add=True)

            @pl.when(i + 2 < nch)
            def _():
                issue_idx(i + 2, b)

            return 0

        lax.fori_loop(0, nch, body, 0)
        plsc.subcore_barrier()

        @pl.when(s < N_SUB - 1)
        def _():
            pltpu.sync_copy(acc_sh.at[pl.ds(r0, rows_sub), :],
                            out_hbm.at[c, pl.ds(r0, rows_sub), :])

        @pl.when(s == N_SUB - 1)
        def _():
            pltpu.sync_copy(acc_sh.at[pl.ds(r0, rows_last), :],
                            out_hbm.at[c, pl.ds(r0, rows_last), :])

    return k


@functools.lru_cache(maxsize=None)
def _deg_kernel(n, e):
    rows_sub, rows_last = _row_share(n)
    acc_n = rows_sub * N_SUB

    @functools.partial(
        pl.kernel,
        out_type=jax.ShapeDtypeStruct((N_CORES, n, 128), jnp.float32),
        mesh=_sc_mesh(),
        scratch_types=[
            pltpu.VMEM((2, CH), jnp.int32),
            pltpu.VMEM((CH, 128), jnp.float32),
            pltpu.VMEM_SHARED((acc_n, 128), jnp.float32),
            pltpu.SemaphoreType.DMA((2,)),
        ],
    )
    def k(dst_hbm, ones_hbm, zeros_hbm, out_hbm, idx_d, ones_v, acc_sh, sem_i):
        c = lax.axis_index("c")
        s = lax.axis_index("s")
        wid = s * N_CORES + c
        r0 = s * rows_sub
        nch = _chunks_for(wid, e)

        def issue_idx(ci, b):
            off = (wid + ci * NW) * CH
            pltpu.async_copy(dst_hbm.at[pl.ds(off, CH)], idx_d.at[b], sem_i.at[b])

        issue_idx(0, 0)
        issue_idx(1, 1)
        pltpu.sync_copy(zeros_hbm, acc_sh.at[pl.ds(r0, rows_sub), :])
        pltpu.sync_copy(ones_hbm, ones_v)
        plsc.subcore_barrier()

        def body(i, _):
            b = lax.rem(i, 2)
            pltpu.make_async_copy(dst_hbm.at[pl.ds(0, CH)], idx_d.at[b],
                                  sem_i.at[b]).wait()
            pltpu.sync_copy(ones_v, acc_sh.at[idx_d.at[b]], add=True)

            @pl.when(i + 2 < nch)
            def _():
                issue_idx(i + 2, b)

            return 0

        lax.fori_loop(0, nch, body, 0)
        plsc.subcore_barrier()

        @pl.when(s < N_SUB - 1)
        def _():
            pltpu.sync_copy(acc_sh.at[pl.ds(r0, rows_sub), :],
                            out_hbm.at[c, pl.ds(r0, rows_sub), :])

        @pl.when(s == N_SUB - 1)
        def _():
            pltpu.sync_copy(acc_sh.at[pl.ds(r0, rows_last), :],
                            out_hbm.at[c, pl.ds(r0, rows_last), :])

    return k


@functools.lru_cache(maxsize=None)
def _pair_sum_kernel(n, e, d):
    assert e // CH >= 2 * NW

    @functools.partial(
        pl.kernel,
        out_type=jax.ShapeDtypeStruct((e, d), jnp.float32),
        mesh=_sc_mesh(),
        scratch_types=[
            pltpu.VMEM((2, CH), jnp.int32),
            pltpu.VMEM((2, CH), jnp.int32),
            pltpu.VMEM((2, CH), jnp.int32),
            pltpu.VMEM((2, CH, d), jnp.float32),
            pltpu.VMEM((2, CH, d), jnp.float32),
            pltpu.VMEM_SHARED((N_SUB * 2 * CH, d), jnp.float32),
            pltpu.SemaphoreType.DMA((2,)),
            pltpu.SemaphoreType.DMA((2,)),
            pltpu.SemaphoreType.DMA((2,)),
            pltpu.SemaphoreType.DMA((2,)),
        ],
    )
    def k(p0_hbm, p1_hbm, ident_hbm, u_hbm, v_hbm, s_hbm,
          idx0, idx1, ident, gu, gv, s_sh, sem_i, sem_gu, sem_gv, sem_st):
        c = lax.axis_index("c")
        s = lax.axis_index("s")
        wid = s * N_CORES + c
        nch = _chunks_for(wid, e)

        def region(b):
            return pl.ds((s * 2 + b) * CH, CH)

        def issue_idx(ci, b):
            off = (wid + ci * NW) * CH
            pltpu.async_copy(p0_hbm.at[pl.ds(off, CH)], idx0.at[b], sem_i.at[b])
            pltpu.async_copy(p1_hbm.at[pl.ds(off, CH)], idx1.at[b], sem_i.at[b])

        def wait_idx(b):
            pltpu.make_async_copy(p0_hbm.at[pl.ds(0, CH)], idx0.at[b],
                                  sem_i.at[b]).wait()
            pltpu.make_async_copy(p1_hbm.at[pl.ds(0, CH)], idx1.at[b],
                                  sem_i.at[b]).wait()

        def issue_gathers(b):
            pltpu.async_copy(u_hbm.at[idx0.at[b]], gu.at[b], sem_gu.at[b])
            pltpu.async_copy(v_hbm.at[idx1.at[b]], gv.at[b], sem_gv.at[b])

        def wait_gathers(b):
            pltpu.make_async_copy(u_hbm.at[pl.ds(0, CH), :], gu.at[b],
                                  sem_gu.at[b]).wait()
            pltpu.make_async_copy(v_hbm.at[pl.ds(0, CH), :], gv.at[b],
                                  sem_gv.at[b]).wait()

        def wait_store(b):
            pltpu.make_async_copy(s_sh.at[region(0), :],
                                  s_hbm.at[pl.ds(0, CH), :], sem_st.at[b]).wait()

        pltpu.sync_copy(ident_hbm.at[pl.ds(s * 2, 2)], ident)
        issue_idx(0, 0)
        issue_idx(1, 1)
        wait_idx(0)
        issue_gathers(0)

        def body(i, _):
            b = lax.rem(i, 2)
            o = 1 - b
            off = (wid + i * NW) * CH
            wait_gathers(b)

            @pl.when(i + 1 < nch)
            def _():
                wait_idx(o)

                @pl.when(i >= 1)
                def _():
                    wait_store(o)

                issue_gathers(o)

            pltpu.sync_copy(gu.at[b], s_sh.at[region(b), :])
            pltpu.sync_copy(gv.at[b], s_sh.at[ident.at[b]], add=True)
            pltpu.async_copy(s_sh.at[region(b), :],
                             s_hbm.at[pl.ds(off, CH), :], sem_st.at[b])

            @pl.when(i + 2 < nch)
            def _():
                issue_idx(i + 2, b)

            return 0

        lax.fori_loop(0, nch, body, 0)
        b_last = lax.rem(nch - 1, 2)
        wait_store(b_last)
        wait_store(1 - b_last)

    return k


def _dinv_of(deg_ref):
    deg = deg_ref[0, :, 0:1] + deg_ref[1, :, 0:1] + 1.0
    return lax.rsqrt(deg)


def _tc_first(degp, x, w, blk=1000):
    n, dfeat = x.shape
    dout = w.shape[1]

    def body(deg_ref, x_ref, w_ref, y_ref):
        y_ref[...] = jnp.dot(x_ref[...], w_ref[...],
                             preferred_element_type=jnp.float32) * _dinv_of(deg_ref)

    return pl.pallas_call(
        body,
        grid=(n // blk,),
        in_specs=[
            pl.BlockSpec((2, blk, 128), lambda i: (0, i, 0)),
            pl.BlockSpec((blk, dfeat), lambda i: (i, 0)),
            pl.BlockSpec((dfeat, dout), lambda i: (0, 0)),
        ],
        out_specs=pl.BlockSpec((blk, dout), lambda i: (i, 0)),
        out_shape=jax.ShapeDtypeStruct((n, dout), jnp.float32),
    )(degp, x, w)


def _tc_mid(degp, acc, y, b, w, blk=1000):
    n, din = y.shape
    dout = w.shape[1]

    def body(deg_ref, acc_ref, y_ref, b_ref, w_ref, o_ref):
        dinv = _dinv_of(deg_ref)
        h = jnp.maximum(dinv * (acc_ref[0] + acc_ref[1] + y_ref[...]) + b_ref[...], 0.0)
        o_ref[...] = jnp.dot(h, w_ref[...],
                             preferred_element_type=jnp.float32) * dinv

    return pl.pallas_call(
        body,
        grid=(n // blk,),
        in_specs=[
            pl.BlockSpec((2, blk, 128), lambda i: (0, i, 0)),
            pl.BlockSpec((2, blk, din), lambda i: (0, i, 0)),
            pl.BlockSpec((blk, din), lambda i: (i, 0)),
            pl.BlockSpec((1, din), lambda i: (0, 0)),
            pl.BlockSpec((din, dout), lambda i: (0, 0)),
        ],
        out_specs=pl.BlockSpec((blk, dout), lambda i: (i, 0)),
        out_shape=jax.ShapeDtypeStruct((n, dout), jnp.float32),
    )(degp, acc, y, b, w)


def _tc_last_node(degp, acc, y, b, w3a, w3b, blk=1000):
    n, din = y.shape
    dout = w3a.shape[1]

    def body(deg_ref, acc_ref, y_ref, b_ref, wa_ref, wb_ref, u_ref, v_ref):
        dinv = _dinv_of(deg_ref)
        h = jnp.maximum(dinv * (acc_ref[0] + acc_ref[1] + y_ref[...]) + b_ref[...], 0.0)
        u_ref[...] = jnp.dot(h, wa_ref[...], preferred_element_type=jnp.float32)
        v_ref[...] = jnp.dot(h, wb_ref[...], preferred_element_type=jnp.float32)

    return pl.pallas_call(
        body,
        grid=(n // blk,),
        in_specs=[
            pl.BlockSpec((2, blk, 128), lambda i: (0, i, 0)),
            pl.BlockSpec((2, blk, din), lambda i: (0, i, 0)),
            pl.BlockSpec((blk, din), lambda i: (i, 0)),
            pl.BlockSpec((1, din), lambda i: (0, 0)),
            pl.BlockSpec((din, dout), lambda i: (0, 0)),
            pl.BlockSpec((din, dout), lambda i: (0, 0)),
        ],
        out_specs=[
            pl.BlockSpec((blk, dout), lambda i: (i, 0)),
            pl.BlockSpec((blk, dout), lambda i: (i, 0)),
        ],
        out_shape=[jax.ShapeDtypeStruct((n, dout), jnp.float32),
                   jax.ShapeDtypeStruct((n, dout), jnp.float32)],
    )(degp, acc, y, b, w3a, w3b)


def _tc_head(s, b3, w4, b4, blk=2000):
    e, d = s.shape

    def body(s_ref, b3_ref, w4_ref, b4_ref, o_ref):
        hid = jnp.maximum(s_ref[...] + b3_ref[...], 0.0)
        logit = jnp.dot(hid, w4_ref[...], preferred_element_type=jnp.float32)
        o_ref[...] = jax.nn.sigmoid(logit + b4_ref[...])

    return pl.pallas_call(
        body,
        grid=(e // blk,),
        in_specs=[
            pl.BlockSpec((blk, d), lambda i: (i, 0)),
            pl.BlockSpec((1, d), lambda i: (0, 0)),
            pl.BlockSpec((d, 1), lambda i: (0, 0)),
            pl.BlockSpec((1, 1), lambda i: (0, 0)),
        ],
        out_specs=pl.BlockSpec((blk, 1), lambda i: (i, 0)),
        out_shape=jax.ShapeDtypeStruct((e, 1), jnp.float32),
    )(s, b3, w4, b4)


def kernel(x, edge_index, edge_pairs, W1, b1, W2, b2, W3, b3, W4, b4):
    x = x.astype(jnp.float32)
    ei = edge_index.astype(jnp.int32)
    ep = edge_pairs.astype(jnp.int32)
    src, dst = ei[0], ei[1]
    p0, p1 = ep[0], ep[1]
    n = x.shape[0]
    e = src.shape[0]
    rows_sub, _ = _row_share(n)
    d1 = W1.shape[1]
    d2 = W2.shape[1]
    dh = W3.shape[1]

    dp = 128
    W1p = jnp.pad(W1, ((0, 0), (0, dp - d1)))
    b1p = jnp.pad(b1, (0, dp - d1))
    W2p = jnp.pad(W2, ((0, dp - d1), (0, 0)))

    ones16 = jnp.ones((CH, 128), jnp.float32)
    z16 = jnp.zeros((rows_sub, 128), jnp.float32)
    z1 = jnp.zeros((rows_sub, dp), jnp.float32)
    z2 = jnp.zeros((rows_sub, d2), jnp.float32)

    ident = jnp.arange(N_SUB * 2 * CH, dtype=jnp.int32).reshape(N_SUB * 2, CH)

    degp = _deg_kernel(n, e)(dst, ones16, z16)
    y1 = _tc_first(degp, x, W1p)
    acc1 = _scatter_add_kernel(n, e, dp)(src, dst, y1, z1)
    y2 = _tc_mid(degp, acc1, y1, b1p.reshape(1, -1), W2p)
    acc2 = _scatter_add_kernel(n, e, d2)(src, dst, y2, z2)
    u, v = _tc_last_node(degp, acc2, y2, b2.reshape(1, -1),
                         W3[:d2], W3[d2:])
    eh = e // 2
    outs = []
    for h in range(2):
        ph0 = lax.dynamic_slice_in_dim(p0, h * eh, eh)
        ph1 = lax.dynamic_slice_in_dim(p1, h * eh, eh)
        s = _pair_sum_kernel(n, eh, dh)(ph0, ph1, ident, u, v)
        outs.append(_tc_head(s, b3.reshape(1, -1), W4, b4.reshape(1, 1)))
    return jnp.concatenate(outs, axis=0)
